# baseline (speedup 1.0000x reference)
import functools

import jax
import jax.numpy as jnp
from jax.experimental import pallas as pl
from jax.experimental.pallas import tpu as pltpu

T = 16384
D = 2048
H = 1024
E = 16
K = 2
TM = 512
NSTEPS = T // TM


def _router_kernel(x_ref, w1_ref, b1_ref, w2_ref, b2_ref, eye_ref,
                   w_ref, i_ref, p_ref, lb_ref):
    eye = eye_ref[...]

    lt = lb_ref[...]
    iota = jax.lax.broadcasted_iota(jnp.int32, (E, TM), 0)
    m1 = jnp.max(lt, axis=0, keepdims=True)
    i1 = jnp.min(jnp.where(lt == m1, iota, E), axis=0, keepdims=True)
    masked = jnp.where(iota == i1, -jnp.inf, lt)
    m2 = jnp.max(masked, axis=0, keepdims=True)
    i2 = jnp.min(jnp.where(masked == m2, iota, E), axis=0, keepdims=True)

    et = jnp.exp(lt - m1)
    zt = jnp.sum(et, axis=0, keepdims=True)
    pt = et / zt
    w1p = jnp.max(pt, axis=0, keepdims=True)
    w2p = jnp.max(jnp.where(iota == i1, -1.0, pt), axis=0, keepdims=True)
    denom = jnp.maximum(w1p + w2p, 1e-6)
    wts = jnp.concatenate([w1p, w2p], axis=0) / denom
    idx = jnp.concatenate([i1, i2], axis=0).astype(jnp.float32)

    tr = lambda a: jax.lax.dot_general(
        eye, a, (((1,), (1,)), ((), ())), preferred_element_type=jnp.float32)
    p_ref[...] = tr(pt)
    w_ref[...] = tr(wts)
    i_ref[...] = tr(idx).astype(jnp.int32)

    x = x_ref[...]
    h = jnp.dot(x, w1_ref[...], preferred_element_type=jnp.float32)
    h = jnp.maximum(h + b1_ref[...], 0.0)
    logits = jnp.dot(h, w2_ref[...], preferred_element_type=jnp.float32)
    logits = logits + b2_ref[...]
    lb_ref[...] = logits.T


@functools.partial(jax.jit, static_argnames=("interpret",))
def kernel(pooled_feat, W1, b1, W2, b2, interpret=False):
    b1r = b1.reshape(1, H)
    b2r = b2.reshape(1, E)
    eye = jnp.eye(TM, dtype=jnp.float32)
    grid = (NSTEPS + 1,)
    out = pl.pallas_call(
        _router_kernel,
        grid=grid,
        in_specs=[
            pl.BlockSpec((TM, D), lambda i: (jnp.minimum(i, NSTEPS - 1), 0)),
            pl.BlockSpec((D, H), lambda i: (0, 0)),
            pl.BlockSpec((1, H), lambda i: (0, 0)),
            pl.BlockSpec((H, E), lambda i: (0, 0)),
            pl.BlockSpec((1, E), lambda i: (0, 0)),
            pl.BlockSpec((TM, TM), lambda i: (0, 0)),
        ],
        out_specs=[
            pl.BlockSpec((TM, K), lambda i: (i, 0)),
            pl.BlockSpec((TM, K), lambda i: (i, 0)),
            pl.BlockSpec((TM, E), lambda i: (i, 0)),
        ],
        out_shape=[
            jax.ShapeDtypeStruct((T + TM, K), jnp.float32),
            jax.ShapeDtypeStruct((T + TM, K), jnp.int32),
            jax.ShapeDtypeStruct((T + TM, E), jnp.float32),
        ],
        scratch_shapes=[pltpu.VMEM((E, TM), jnp.float32)],
        compiler_params=pltpu.CompilerParams(
            dimension_semantics=("arbitrary",)),
        interpret=interpret,
    )(pooled_feat, W1, b1r, W2, b2r, eye)
    return (out[0][TM:], out[1][TM:], out[2][TM:])


# speedup vs baseline: 1.0702x; 1.0702x over previous
import functools

import jax
import jax.numpy as jnp
from jax.experimental import pallas as pl
from jax.experimental.pallas import tpu as pltpu

T = 16384
D = 2048
H = 1024
E = 16
K = 2
TM = 512
NSTEPS = T // TM


def _router_kernel(x_ref, w1_ref, b1_ref, w2_ref, b2_ref, eye_ref,
                   w_ref, i_ref, p_ref):
    eye = eye_ref[...]

    x = x_ref[...]
    h = jnp.dot(x, w1_ref[...], preferred_element_type=jnp.float32)
    h = jnp.maximum(h + b1_ref[...], 0.0)
    logits = jnp.dot(h, w2_ref[...], preferred_element_type=jnp.float32)
    logits = logits + b2_ref[...]

    lt = logits.T
    iota = jax.lax.broadcasted_iota(jnp.int32, (E, TM), 0)
    m1 = jnp.max(lt, axis=0, keepdims=True)
    i1 = jnp.min(jnp.where(lt == m1, iota, E), axis=0, keepdims=True)
    masked = jnp.where(iota == i1, -jnp.inf, lt)
    m2 = jnp.max(masked, axis=0, keepdims=True)
    i2 = jnp.min(jnp.where(masked == m2, iota, E), axis=0, keepdims=True)

    et = jnp.exp(lt - m1)
    zt = jnp.sum(et, axis=0, keepdims=True)
    pt = et / zt
    w1p = jnp.max(pt, axis=0, keepdims=True)
    w2p = jnp.max(jnp.where(iota == i1, -1.0, pt), axis=0, keepdims=True)
    denom = jnp.maximum(w1p + w2p, 1e-6)
    wts = jnp.concatenate([w1p, w2p], axis=0) / denom
    idx = jnp.concatenate([i1, i2], axis=0).astype(jnp.float32)

    tr = lambda a: jax.lax.dot_general(
        eye, a, (((1,), (1,)), ((), ())), preferred_element_type=jnp.float32)
    p_ref[...] = tr(pt)
    w_ref[...] = tr(wts)
    i_ref[...] = tr(idx).astype(jnp.int32)


@functools.partial(jax.jit, static_argnames=("interpret",))
def kernel(pooled_feat, W1, b1, W2, b2, interpret=False):
    b1r = b1.reshape(1, H)
    b2r = b2.reshape(1, E)
    eye = jnp.eye(TM, dtype=jnp.float32)
    grid = (NSTEPS,)
    out = pl.pallas_call(
        _router_kernel,
        grid=grid,
        in_specs=[
            pl.BlockSpec((TM, D), lambda i: (i, 0)),
            pl.BlockSpec((D, H), lambda i: (0, 0)),
            pl.BlockSpec((1, H), lambda i: (0, 0)),
            pl.BlockSpec((H, E), lambda i: (0, 0)),
            pl.BlockSpec((1, E), lambda i: (0, 0)),
            pl.BlockSpec((TM, TM), lambda i: (0, 0)),
        ],
        out_specs=[
            pl.BlockSpec((TM, K), lambda i: (i, 0)),
            pl.BlockSpec((TM, K), lambda i: (i, 0)),
            pl.BlockSpec((TM, E), lambda i: (i, 0)),
        ],
        out_shape=[
            jax.ShapeDtypeStruct((T, K), jnp.float32),
            jax.ShapeDtypeStruct((T, K), jnp.int32),
            jax.ShapeDtypeStruct((T, E), jnp.float32),
        ],
        compiler_params=pltpu.CompilerParams(
            dimension_semantics=("parallel",)),
        interpret=interpret,
    )(pooled_feat, W1, b1r, W2, b2r, eye)
    return (out[0], out[1], out[2])
